# Initial kernel scaffold; baseline (speedup 1.0000x reference)
import jax
import jax.numpy as jnp
from jax.experimental import pallas as pl


def kernel(x, weight, bias, gamma, beta):
    raise NotImplementedError("write your pallas kernel here")


# speedup vs baseline: 1.0612x; 1.0612x over previous
import functools

import jax
import jax.numpy as jnp
from jax import lax
from jax.experimental import pallas as pl
from jax.experimental.pallas import tpu as pltpu


def _stats_kernel(x_ref, g_ref, s_ref):
    i = pl.program_id(1)

    @pl.when(i == 0)
    def _init():
        g_ref[...] = jnp.zeros_like(g_ref)
        s_ref[...] = jnp.zeros_like(s_ref)

    xt = x_ref[0]
    xb = xt.astype(jnp.bfloat16)
    g_ref[0] += lax.dot_general(
        xb, xb, (((1,), (1,)), ((), ())), preferred_element_type=jnp.float32)
    s_ref[0] += jnp.sum(xb.astype(jnp.float32), axis=1, keepdims=True)


def _apply_kernel(x_ref, w_ref, shift_ref, o_ref):
    xb = x_ref[0].astype(jnp.bfloat16)
    y = lax.dot_general(
        w_ref[...], xb, (((1,), (0,)), ((), ())),
        preferred_element_type=jnp.float32)
    o_ref[0] = jnp.maximum(y + shift_ref[...], 0.0)


def kernel(x, weight, bias, gamma, beta):
    del bias
    eps = 1e-5
    N, Cin, H, W = x.shape
    _, Cout, KH, KW = weight.shape
    KK = KH * KW
    J = KK * Cout
    HW = H * W
    M = N * HW
    inv_count = 1.0 / float(M * KK)

    x3 = x.reshape(N, Cin, HW)
    w_t = jnp.transpose(weight, (2, 3, 1, 0)).reshape(J, Cin)

    cores = 2
    n_per = N // cores
    g2, s2 = pl.pallas_call(
        _stats_kernel,
        out_shape=(jax.ShapeDtypeStruct((cores, Cin, Cin), jnp.float32),
                   jax.ShapeDtypeStruct((cores, Cin, 1), jnp.float32)),
        grid=(cores, n_per),
        in_specs=[pl.BlockSpec((1, Cin, HW),
                               lambda c, i: (c * n_per + i, 0, 0))],
        out_specs=(pl.BlockSpec((1, Cin, Cin), lambda c, i: (c, 0, 0)),
                   pl.BlockSpec((1, Cin, 1), lambda c, i: (c, 0, 0))),
        compiler_params=pltpu.CompilerParams(
            dimension_semantics=("parallel", "arbitrary")),
    )(x3)

    g = g2.sum(0)
    s = s2.sum(0)[:, 0]
    wb32 = w_t.astype(jnp.bfloat16).astype(jnp.float32)
    wsum = wb32.reshape(KK, Cout, Cin).sum(0)
    mean_c = (wsum @ s) * inv_count
    e2_c = jnp.sum((wb32 @ g) * wb32, axis=1).reshape(KK, Cout).sum(0) * inv_count
    var_c = e2_c - mean_c * mean_c
    scale_c = gamma.astype(jnp.float32) * lax.rsqrt(var_c + eps)
    shift_c = beta.astype(jnp.float32) - mean_c * scale_c
    wsb = (w_t.reshape(KK, Cout, Cin) * scale_c[None, :, None]
           ).reshape(J, Cin).astype(jnp.bfloat16)
    shift_j = jnp.tile(shift_c, KK).reshape(J, 1)

    TM = min(1024, HW)
    T = HW // TM
    OH, OW = KH * H, KW * W
    o5 = pl.pallas_call(
        _apply_kernel,
        out_shape=jax.ShapeDtypeStruct((N, J, HW), jnp.float32),
        grid=(N, T),
        in_specs=[
            pl.BlockSpec((1, Cin, TM), lambda n, t: (n, 0, t)),
            pl.BlockSpec((J, Cin), lambda n, t: (0, 0)),
            pl.BlockSpec((J, 1), lambda n, t: (0, 0)),
        ],
        out_specs=pl.BlockSpec((1, J, TM), lambda n, t: (n, 0, t)),
        compiler_params=pltpu.CompilerParams(
            dimension_semantics=("parallel", "arbitrary")),
    )(x3, wsb, shift_j)
    out = o5.reshape(N, KH, KW, Cout, H, W).transpose(0, 3, 4, 1, 5, 2)
    return out.reshape(N, Cout, OH, OW)


# speedup vs baseline: 2.3219x; 2.1879x over previous
import functools

import jax
import jax.numpy as jnp
from jax import lax
from jax.experimental import pallas as pl
from jax.experimental.pallas import tpu as pltpu


def _stats_kernel(x_ref, w_ref, gam_ref, bet_ref, wsb_ref, shift_ref,
                  g_sc, s_sc, *, n_imgs, kk, inv_count, eps):
    i = pl.program_id(0)

    @pl.when(i == 0)
    def _init():
        g_sc[...] = jnp.zeros_like(g_sc)
        s_sc[...] = jnp.zeros_like(s_sc)

    xb = x_ref[0].astype(jnp.bfloat16)
    g_sc[...] += lax.dot_general(
        xb, xb, (((1,), (1,)), ((), ())), preferred_element_type=jnp.float32)
    s_sc[...] += jnp.sum(xb.astype(jnp.float32), axis=1, keepdims=True)

    @pl.when(i == n_imgs - 1)
    def _epilogue():
        j, cin = w_ref.shape
        cout = j // kk
        g = g_sc[...]
        s = s_sc[...]
        wt = w_ref[...]
        wb32 = wt.astype(jnp.bfloat16).astype(jnp.float32)
        wsum = jnp.sum(wb32.reshape(kk, cout, cin), axis=0)
        mean = jnp.dot(wsum, s,
                       preferred_element_type=jnp.float32) * inv_count
        p = jnp.dot(wb32, g, preferred_element_type=jnp.float32)
        e2j = jnp.sum(p * wb32, axis=1, keepdims=True)
        e2 = jnp.sum(e2j.reshape(kk, cout, 1), axis=0) * inv_count
        var = e2 - mean * mean
        scale = gam_ref[...] * lax.rsqrt(var + eps)
        shift_ref[...] = bet_ref[...] - mean * scale
        wsb_ref[...] = (wt.reshape(kk, cout, cin) * scale[None]
                        ).reshape(j, cin).astype(jnp.bfloat16)


def _apply_kernel(x_ref, w_ref, d_ref, shift_ref, o_ref, *, cout, h_per_blk,
                  ow_len):
    xb = x_ref[0].astype(jnp.bfloat16)
    lane = lax.broadcasted_iota(jnp.int32, (cout, ow_len), 1)
    even = lane % 2 == 0
    shift = shift_ref[...]
    xd = jnp.dot(xb, d_ref[...],
                 preferred_element_type=jnp.float32)
    xdb = xd.astype(jnp.bfloat16)
    y = jnp.dot(w_ref[...], xdb,
                preferred_element_type=jnp.float32)
    for h in range(h_per_blk):
        for kh in range(2):
            c0, c1 = h * ow_len, (h + 1) * ow_len
            z0 = y[(kh * 2) * cout:(kh * 2 + 1) * cout, c0:c1]
            z1 = y[(kh * 2 + 1) * cout:(kh * 2 + 2) * cout, c0:c1]
            c = jnp.maximum(jnp.where(even, z0, z1) + shift, 0.0)
            o_ref[:, 2 * h + kh, :] = c


def kernel(x, weight, bias, gamma, beta):
    del bias
    eps = 1e-5
    N, Cin, H, W = x.shape
    _, Cout, KH, KW = weight.shape
    KK = KH * KW
    J = KK * Cout
    HW = H * W
    M = N * HW
    inv_count = 1.0 / float(M * KK)

    x3 = x.reshape(N, Cin, HW)
    w_t = jnp.transpose(weight, (2, 3, 1, 0)).reshape(J, Cin)

    wsb, shift_col = pl.pallas_call(
        functools.partial(_stats_kernel, n_imgs=N, kk=KK,
                          inv_count=inv_count, eps=eps),
        out_shape=(jax.ShapeDtypeStruct((J, Cin), jnp.bfloat16),
                   jax.ShapeDtypeStruct((Cout, 1), jnp.float32)),
        grid=(N,),
        in_specs=[
            pl.BlockSpec((1, Cin, HW), lambda i: (i, 0, 0)),
            pl.BlockSpec((J, Cin), lambda i: (0, 0)),
            pl.BlockSpec((Cout, 1), lambda i: (0, 0)),
            pl.BlockSpec((Cout, 1), lambda i: (0, 0)),
        ],
        out_specs=(pl.BlockSpec((J, Cin), lambda i: (0, 0)),
                   pl.BlockSpec((Cout, 1), lambda i: (0, 0))),
        scratch_shapes=[pltpu.VMEM((Cin, Cin), jnp.float32),
                        pltpu.VMEM((Cin, 1), jnp.float32)],
        compiler_params=pltpu.CompilerParams(
            dimension_semantics=("arbitrary",)),
    )(x3, w_t, gamma.reshape(Cout, 1), beta.reshape(Cout, 1))

    OH, OW = KH * H, KW * W
    h_per_blk = 4
    blk = h_per_blk * W
    m_idx = jnp.arange(blk)[:, None]
    col = jnp.arange(h_per_blk * OW)[None, :]
    dmat = ((m_idx // W == col // OW) & (m_idx % W == (col % OW) // KW)
            ).astype(jnp.bfloat16)

    rows_per_blk = 2 * h_per_blk
    out = pl.pallas_call(
        functools.partial(_apply_kernel, cout=Cout, h_per_blk=h_per_blk,
                          ow_len=OW),
        out_shape=jax.ShapeDtypeStruct((N * Cout, OH, OW), jnp.float32),
        grid=(N, OH // rows_per_blk),
        in_specs=[
            pl.BlockSpec((1, Cin, blk), lambda n, t: (n, 0, t)),
            pl.BlockSpec((J, Cin), lambda n, t: (0, 0)),
            pl.BlockSpec((blk, h_per_blk * OW), lambda n, t: (0, 0)),
            pl.BlockSpec((Cout, 1), lambda n, t: (0, 0)),
        ],
        out_specs=pl.BlockSpec((Cout, rows_per_blk, OW),
                               lambda n, t: (n, t, 0)),
        compiler_params=pltpu.CompilerParams(
            dimension_semantics=("parallel", "arbitrary")),
    )(x3, wsb, dmat, shift_col)
    return out.reshape(N, Cout, OH, OW)
